# speedup vs baseline: 1.0316x; 1.0316x over previous
import functools

import jax
import jax.numpy as jnp
from jax.experimental import pallas as pl

N_SUB = 10000
D = 128
ROW_BLK = 1000


def _hop(x, W, b, s, o):
    h = jax.lax.dot_general(x, W, (((1,), (1,)), ((), ())),
                            preferred_element_type=jnp.float32)
    h = jax.nn.relu(h + b)
    mean = jnp.mean(h, axis=1, keepdims=True)
    var = jnp.mean((h - mean) ** 2, axis=1, keepdims=True) + 1e-9
    return (h - mean) * s * jax.lax.rsqrt(var) + o


def _layer1_body(x_ref, h1_ref, W0_ref, b0_ref, s0_ref, o0_ref,
                 W1_ref, b1_ref, s1_ref, o1_ref, out_ref):
    x = x_ref[...]
    h1 = h1_ref[...]
    out_ref[...] = (_hop(x, W0_ref[...], b0_ref[...], s0_ref[...], o0_ref[...])
                    + _hop(h1, W1_ref[...], b1_ref[...], s1_ref[...], o1_ref[...]))


def _layer2_body(x_ref, h1_ref, W0_ref, b0_ref, s0_ref, o0_ref,
                 W1_ref, b1_ref, s1_ref, o1_ref, Wc_ref, bc_ref, out_ref):
    x = x_ref[...]
    h1 = h1_ref[...]
    x2 = (_hop(x, W0_ref[...], b0_ref[...], s0_ref[...], o0_ref[...])
          + _hop(h1, W1_ref[...], b1_ref[...], s1_ref[...], o1_ref[...]))
    nrm = jnp.sqrt(jnp.sum(x2 * x2, axis=1, keepdims=True))
    x2 = x2 / jnp.maximum(nrm, 1e-12)
    out_ref[...] = jax.lax.dot_general(x2, Wc_ref[...], (((1,), (1,)), ((), ())),
                                       preferred_element_type=jnp.float32) + bc_ref[...]


_row_spec = pl.BlockSpec((ROW_BLK, D), lambda i: (i, 0))
_mat_spec = pl.BlockSpec((D, D), lambda i: (0, 0))
_vec_spec = pl.BlockSpec((1, D), lambda i: (0, 0))


def _dense_layer1(x, h1, W0, b0, s0, o0, W1, b1, s1, o1):
    return pl.pallas_call(
        _layer1_body,
        grid=(N_SUB // ROW_BLK,),
        in_specs=[_row_spec, _row_spec,
                  _mat_spec, _vec_spec, _vec_spec, _vec_spec,
                  _mat_spec, _vec_spec, _vec_spec, _vec_spec],
        out_specs=_row_spec,
        out_shape=jax.ShapeDtypeStruct((N_SUB, D), jnp.float32),
    )(x, h1, W0, b0, s0, o0, W1, b1, s1, o1)


def _dense_layer2(x, h1, W0, b0, s0, o0, W1, b1, s1, o1, Wc, bc):
    return pl.pallas_call(
        _layer2_body,
        grid=(N_SUB // ROW_BLK,),
        in_specs=[_row_spec, _row_spec,
                  _mat_spec, _vec_spec, _vec_spec, _vec_spec,
                  _mat_spec, _vec_spec, _vec_spec, _vec_spec,
                  _mat_spec, _vec_spec],
        out_specs=_row_spec,
        out_shape=jax.ShapeDtypeStruct((N_SUB, D), jnp.float32),
    )(x, h1, W0, b0, s0, o0, W1, b1, s1, o1, Wc, bc)


def kernel(node_subgraph, edge_index, edge_val, feat_full,
           W1_0, b1_0, sc1_0, off1_0, W1_1, b1_1, sc1_1, off1_1,
           W2_0, b2_0, sc2_0, off2_0, W2_1, b2_1, sc2_1, off2_1,
           Wc, bc):
    r2 = lambda v: v.reshape(1, D)
    dst = edge_index[0]
    src = edge_index[1]

    x0 = feat_full[node_subgraph]
    h1 = jnp.zeros((N_SUB, D), jnp.float32).at[dst].add(
        x0[src] * edge_val[:, None])
    x1 = _dense_layer1(x0, h1,
                       W1_0, r2(b1_0), r2(sc1_0), r2(off1_0),
                       W1_1, r2(b1_1), r2(sc1_1), r2(off1_1))
    h2 = jnp.zeros((N_SUB, D), jnp.float32).at[dst].add(
        x1[src] * edge_val[:, None])
    pred = _dense_layer2(x1, h2,
                         W2_0, r2(b2_0), r2(sc2_0), r2(off2_0),
                         W2_1, r2(b2_1), r2(sc2_1), r2(off2_1),
                         Wc, r2(bc))
    return pred


# speedup vs baseline: 10.6582x; 10.3315x over previous
import functools

import jax
import jax.numpy as jnp
from jax import lax
from jax.experimental import pallas as pl
from jax.experimental.pallas import tpu as pltpu
from jax.experimental.pallas import tpu_sc as plsc

N_SUB = 10000
D = 128
E = 320000
ROW_BLK = 1000

NC = 2
NS = 16
NW = NC * NS
EPW = E // NW
K = 125
NCH = EPW // K
CP = 632

_sc_mesh = plsc.VectorSubcoreMesh(
    core_axis_name="c", subcore_axis_name="s", num_cores=NC, num_subcores=NS)


GK = 80
GNCH = N_SUB // GK


@functools.partial(
    pl.kernel,
    out_type=jax.ShapeDtypeStruct((N_SUB, D), jnp.float32),
    mesh=_sc_mesh,
    compiler_params=pltpu.CompilerParams(needs_layout_passes=False),
    scratch_types=[
        pltpu.VMEM((GK,), jnp.int32),
        pltpu.VMEM((GK, D), jnp.float32),
        pltpu.SemaphoreType.DMA,
    ],
)
def _sc_gather(idx_hbm, table_hbm, out_hbm, idx_v, rows_v, sem):
    c = lax.axis_index("c")
    s = lax.axis_index("s")
    wid = c * NS + s
    for k in range(4):
        chunk = wid + NW * k

        @pl.when(chunk < GNCH)
        def _():
            base = chunk * GK
            pltpu.sync_copy(idx_hbm.at[pl.ds(base, GK)], idx_v)
            pltpu.async_copy(table_hbm.at[idx_v], rows_v, sem).wait()
            pltpu.sync_copy(rows_v, out_hbm.at[pl.ds(base, GK)])


def _bc16(v):
    return jnp.full((16,), v, jnp.int32)


@functools.partial(
    pl.kernel,
    out_type=jax.ShapeDtypeStruct((NC, N_SUB, D), jnp.float32),
    mesh=_sc_mesh,
    compiler_params=pltpu.CompilerParams(needs_layout_passes=False),
    scratch_types=[
        [pltpu.VMEM((K,), jnp.int32)] * 3,
        [pltpu.VMEM((K,), jnp.float32)] * 3,
        [pltpu.VMEM((K,), jnp.int32)] * 3,
        [pltpu.VMEM((K, D), jnp.float32)] * 3,
        pltpu.VMEM_SHARED((N_SUB, D), jnp.float32),
        [pltpu.SemaphoreType.DMA] * 3,
        [pltpu.SemaphoreType.DMA] * 3,
        [pltpu.SemaphoreType.DMA] * 3,
        [pltpu.SemaphoreType.DMA] * 3,
        [pltpu.SemaphoreType.DMA] * 3,
    ],
)
def _sc_spmm(dst_hbm, src_hbm, val_hbm, x_hbm, zeros_hbm, out_hbm,
             srcs, vals, dsts, rows, acc, gsems, ssems, csems, vsems, dsems):
    c = lax.axis_index("c")
    s = lax.axis_index("s")
    wid = c * NS + s

    start = pl.multiple_of(s * (N_SUB // NS) - lax.rem(s, 8), 8)

    pltpu.sync_copy(zeros_hbm, acc.at[pl.ds(start, CP)])

    plsc.subcore_barrier()

    def _scale(rbuf, vbuf):
        @plsc.parallel_loop(0, K, step=1, unroll=5)
        def body(e):
            vb = plsc.load_gather(vbuf, [_bc16(e)])
            for q in range(8):
                sl = pl.ds(q * 16, 16)
                rbuf[e, sl] = rbuf[e, sl] * vb

    def _stage_idx(i, b):
        pltpu.async_copy(src_hbm.at[wid, i], srcs[b], csems[b])
        pltpu.async_copy(val_hbm.at[wid, i], vals[b], vsems[b])

    def _launch_gather(i, b):
        pltpu.async_copy(dst_hbm.at[wid, i], dsts[b], dsems[b])
        pltpu.make_async_copy(src_hbm.at[wid, i], srcs[b], csems[b]).wait()
        pltpu.make_async_copy(x_hbm.at[srcs[b]], rows[b], gsems[b]).start()

    def _body(j, b):
        bp2 = (b + 2) % 3
        pltpu.make_async_copy(x_hbm.at[srcs[b]], rows[b], gsems[b]).wait()
        pltpu.make_async_copy(val_hbm.at[wid, j], vals[b], vsems[b]).wait()
        _scale(rows[b], vals[b])
        pltpu.make_async_copy(dst_hbm.at[wid, j], dsts[b], dsems[b]).wait()
        pltpu.async_copy(rows[b], acc.at[dsts[b]], ssems[b], add=True)

        pl.when(j >= 1)(
            lambda: pltpu.make_async_copy(
                rows[bp2], acc.at[dsts[bp2]], ssems[bp2]).wait())
        pl.when(j + 3 < NCH)(lambda: _stage_idx(j + 3, b))
        pl.when(j + 2 < NCH)(lambda: _launch_gather(j + 2, bp2))

    for i in range(3):
        _stage_idx(i, i)
    for i in range(2):
        _launch_gather(i, i)

    def outer(t, _):
        for u in range(3):
            j = 3 * t + u
            pl.when(j < NCH)(lambda jj=j, bb=u: _body(jj, bb))
        return 0

    lax.fori_loop(0, (NCH + 2) // 3, outer, 0)
    bl = (NCH - 1) % 3
    pltpu.make_async_copy(rows[bl], acc.at[dsts[bl]], ssems[bl]).wait()

    plsc.subcore_barrier()

    pltpu.sync_copy(acc.at[pl.ds(start, CP)],
                    out_hbm.at[c].at[pl.ds(start, CP)])


def _hop(x, W, b, s, o):
    h = jax.lax.dot_general(x, W, (((1,), (1,)), ((), ())),
                            preferred_element_type=jnp.float32)
    h = jax.nn.relu(h + b)
    mean = jnp.mean(h, axis=1, keepdims=True)
    var = jnp.mean((h - mean) ** 2, axis=1, keepdims=True) + 1e-9
    return (h - mean) * s * jax.lax.rsqrt(var) + o


def _layer1_body(x_ref, p_ref, W0_ref, b0_ref, s0_ref, o0_ref,
                 W1_ref, b1_ref, s1_ref, o1_ref, out_ref):
    x = x_ref[...]
    h1 = p_ref[0] + p_ref[1]
    out_ref[...] = (_hop(x, W0_ref[...], b0_ref[...], s0_ref[...], o0_ref[...])
                    + _hop(h1, W1_ref[...], b1_ref[...], s1_ref[...], o1_ref[...]))


def _layer2_body(x_ref, p_ref, W0_ref, b0_ref, s0_ref, o0_ref,
                 W1_ref, b1_ref, s1_ref, o1_ref, Wc_ref, bc_ref, out_ref):
    x = x_ref[...]
    h1 = p_ref[0] + p_ref[1]
    x2 = (_hop(x, W0_ref[...], b0_ref[...], s0_ref[...], o0_ref[...])
          + _hop(h1, W1_ref[...], b1_ref[...], s1_ref[...], o1_ref[...]))
    nrm = jnp.sqrt(jnp.sum(x2 * x2, axis=1, keepdims=True))
    x2 = x2 / jnp.maximum(nrm, 1e-12)
    out_ref[...] = jax.lax.dot_general(x2, Wc_ref[...], (((1,), (1,)), ((), ())),
                                       preferred_element_type=jnp.float32) + bc_ref[...]


_row_spec = pl.BlockSpec((ROW_BLK, D), lambda i: (i, 0))
_par_spec = pl.BlockSpec((NC, ROW_BLK, D), lambda i: (0, i, 0))
_mat_spec = pl.BlockSpec((D, D), lambda i: (0, 0))
_vec_spec = pl.BlockSpec((1, D), lambda i: (0, 0))


def _dense_layer1(x, p, W0, b0, s0, o0, W1, b1, s1, o1):
    return pl.pallas_call(
        _layer1_body,
        grid=(N_SUB // ROW_BLK,),
        in_specs=[_row_spec, _par_spec,
                  _mat_spec, _vec_spec, _vec_spec, _vec_spec,
                  _mat_spec, _vec_spec, _vec_spec, _vec_spec],
        out_specs=_row_spec,
        out_shape=jax.ShapeDtypeStruct((N_SUB, D), jnp.float32),
    )(x, p, W0, b0, s0, o0, W1, b1, s1, o1)


def _dense_layer2(x, p, W0, b0, s0, o0, W1, b1, s1, o1, Wc, bc):
    return pl.pallas_call(
        _layer2_body,
        grid=(N_SUB // ROW_BLK,),
        in_specs=[_row_spec, _par_spec,
                  _mat_spec, _vec_spec, _vec_spec, _vec_spec,
                  _mat_spec, _vec_spec, _vec_spec, _vec_spec,
                  _mat_spec, _vec_spec],
        out_specs=_row_spec,
        out_shape=jax.ShapeDtypeStruct((N_SUB, D), jnp.float32),
    )(x, p, W0, b0, s0, o0, W1, b1, s1, o1, Wc, bc)


def kernel(node_subgraph, edge_index, edge_val, feat_full,
           W1_0, b1_0, sc1_0, off1_0, W1_1, b1_1, sc1_1, off1_1,
           W2_0, b2_0, sc2_0, off2_0, W2_1, b2_1, sc2_1, off2_1,
           Wc, bc):
    r2 = lambda v: v.reshape(1, D)
    dst3 = edge_index[0].reshape(NW, NCH, K)
    src3 = edge_index[1].reshape(NW, NCH, K)
    val3 = edge_val.reshape(NW, NCH, K)
    zeros = jnp.zeros((CP, D), jnp.float32)

    x0 = _sc_gather(node_subgraph, feat_full)
    p1 = _sc_spmm(dst3, src3, val3, x0, zeros)
    x1 = _dense_layer1(x0, p1,
                       W1_0, r2(b1_0), r2(sc1_0), r2(off1_0),
                       W1_1, r2(b1_1), r2(sc1_1), r2(off1_1))
    p2 = _sc_spmm(dst3, src3, val3, x1, zeros)
    pred = _dense_layer2(x1, p2,
                         W2_0, r2(b2_0), r2(sc2_0), r2(off2_0),
                         W2_1, r2(b2_1), r2(sc2_1), r2(off2_1),
                         Wc, r2(bc))
    return pred
